# Initial kernel scaffold; baseline (speedup 1.0000x reference)
import functools

import jax
import jax.numpy as jnp
from jax import lax
from jax.experimental import pallas as pl
from jax.experimental.pallas import tpu as pltpu
from jax.experimental.pallas import tpu_sc as plsc

H = 128
HH = H // 2
N = 16384
E = 65536
B = 16
NC, NS, LANES = 2, 16, 16
NW = NC * NS

F32 = jnp.float32
HIGH = lax.Precision.HIGHEST


def _dotT(a, b):
    return jnp.dot(a, b, preferred_element_type=F32, precision=HIGH)


def _dot_c0(a, b):
    return lax.dot_general(a, b, (((0,), (0,)), ((), ())),
                           preferred_element_type=F32, precision=HIGH)


ROWS_PER_W = N // NW
GCH = 128
GITER = ROWS_PER_W // GCH

_SC_MESH = plsc.VectorSubcoreMesh(core_axis_name="c", subcore_axis_name="s",
                                  num_cores=NC, num_subcores=NS)


@functools.partial(
    pl.kernel,
    out_type=jax.ShapeDtypeStruct((N, H), F32),
    mesh=_SC_MESH,
    scratch_types=[
        pltpu.VMEM((GITER, GCH), jnp.int32),
        pltpu.VMEM((GCH, H), F32),
        pltpu.SemaphoreType.DMA,
    ],
)
def _sc_gather(table_hbm, idx_hbm, out_hbm, idx_v, rows_v, sem):
    wid = lax.axis_index("s") * NC + lax.axis_index("c")
    base = wid * ROWS_PER_W
    pltpu.sync_copy(idx_hbm.at[pl.ds(wid * GITER, GITER)], idx_v)
    for j in range(GITER):
        pltpu.async_copy(table_hbm.at[idx_v.at[j]], rows_v, sem).wait()
        pltpu.sync_copy(rows_v, out_hbm.at[pl.ds(base + j * GCH, GCH)])


E_PER_TILE = E // NS
ECH = 128
EITER = E_PER_TILE // ECH
AGG_ROWS_PER_TILE = N // NS


@functools.partial(
    pl.kernel,
    out_type=(jax.ShapeDtypeStruct((N, HH), F32),
              jax.ShapeDtypeStruct((N, HH), F32)),
    mesh=_SC_MESH,
    scratch_types=[
        pltpu.VMEM((ECH,), jnp.int32),
        pltpu.VMEM((ECH,), jnp.int32),
        pltpu.VMEM((ECH, HH), F32),
        pltpu.VMEM_SHARED((N, HH), F32),
        pltpu.SemaphoreType.DMA,
    ],
)
def _sc_edge_agg(mlo_hbm, mhi_hbm, src_hbm, dst_hbm, outlo_hbm, outhi_hbm,
                 srcv, dstv, rows_v, agg_sh, sem):
    c = lax.axis_index("c")
    s = lax.axis_index("s")

    def _zrow(i, carry):
        for q in range(HH // LANES):
            rows_v[i, pl.ds(q * LANES, LANES)] = jnp.zeros((LANES,), F32)
        return carry
    lax.fori_loop(0, ECH, _zrow, 0)
    for j in range(AGG_ROWS_PER_TILE // ECH):
        pltpu.sync_copy(
            rows_v, agg_sh.at[pl.ds(s * AGG_ROWS_PER_TILE + j * ECH, ECH)])
    plsc.subcore_barrier()

    def _edge_chunk(i, carry):
        off = s * E_PER_TILE + i * ECH
        pltpu.sync_copy(src_hbm.at[pl.ds(off, ECH)], srcv)
        pltpu.sync_copy(dst_hbm.at[pl.ds(off, ECH)], dstv)

        @pl.when(c == 0)
        def _():
            pltpu.async_copy(mlo_hbm.at[srcv], rows_v, sem).wait()

        @pl.when(c == 1)
        def _():
            pltpu.async_copy(mhi_hbm.at[srcv], rows_v, sem).wait()

        pltpu.sync_copy(rows_v, agg_sh.at[dstv], add=True)
        return carry
    lax.fori_loop(0, EITER, _edge_chunk, 0)
    plsc.subcore_barrier()

    @pl.when(c == 0)
    def _():
        pltpu.sync_copy(agg_sh.at[pl.ds(s * AGG_ROWS_PER_TILE,
                                        AGG_ROWS_PER_TILE)],
                        outlo_hbm.at[pl.ds(s * AGG_ROWS_PER_TILE,
                                           AGG_ROWS_PER_TILE)])

    @pl.when(c == 1)
    def _():
        pltpu.sync_copy(agg_sh.at[pl.ds(s * AGG_ROWS_PER_TILE,
                                        AGG_ROWS_PER_TILE)],
                        outhi_hbm.at[pl.ds(s * AGG_ROWS_PER_TILE,
                                           AGG_ROWS_PER_TILE)])


MBLK = 2048


def _tc_m_body(h_ref, w_ref, mlo_ref, mhi_ref):
    m = _dotT(h_ref[...], w_ref[...])
    mlo_ref[...] = m[:, :HH]
    mhi_ref[...] = m[:, HH:]


def _tc_m(h, w):
    return pl.pallas_call(
        _tc_m_body,
        grid=(N // MBLK,),
        in_specs=[pl.BlockSpec((MBLK, H), lambda i: (i, 0)),
                  pl.BlockSpec((H, H), lambda i: (0, 0))],
        out_specs=[pl.BlockSpec((MBLK, HH), lambda i: (i, 0)),
                   pl.BlockSpec((MBLK, HH), lambda i: (i, 0))],
        out_shape=[jax.ShapeDtypeStruct((N, HH), F32),
                   jax.ShapeDtypeStruct((N, HH), F32)],
    )(h, w)


def _gru_math(h, agg, wihT, whhT, bih, bhh):
    gi = _dotT(agg, wihT) + bih
    gh = _dotT(h, whhT) + bhh
    i_r, i_z, i_n = gi[:, :H], gi[:, H:2 * H], gi[:, 2 * H:]
    h_r, h_z, h_n = gh[:, :H], gh[:, H:2 * H], gh[:, 2 * H:]
    r = jax.nn.sigmoid(i_r + h_r)
    z = jax.nn.sigmoid(i_z + h_z)
    n = jnp.tanh(i_n + r * h_n)
    return (1.0 - z) * n + z * h


def _tc_gru_m_body(h_ref, alo_ref, ahi_ref, wihT_ref, whhT_ref, bih_ref,
                   bhh_ref, wnext_ref, hout_ref, mlo_ref, mhi_ref):
    agg = jnp.concatenate([alo_ref[...], ahi_ref[...]], axis=1)
    hn = _gru_math(h_ref[...], agg, wihT_ref[...], whhT_ref[...],
                   bih_ref[...], bhh_ref[...])
    hout_ref[...] = hn
    m = _dotT(hn, wnext_ref[...])
    mlo_ref[...] = m[:, :HH]
    mhi_ref[...] = m[:, HH:]


def _tc_gru_m(h, alo, ahi, wihT, whhT, bih, bhh, wnext):
    return pl.pallas_call(
        _tc_gru_m_body,
        grid=(N // MBLK,),
        in_specs=[pl.BlockSpec((MBLK, H), lambda i: (i, 0)),
                  pl.BlockSpec((MBLK, HH), lambda i: (i, 0)),
                  pl.BlockSpec((MBLK, HH), lambda i: (i, 0)),
                  pl.BlockSpec((H, 3 * H), lambda i: (0, 0)),
                  pl.BlockSpec((H, 3 * H), lambda i: (0, 0)),
                  pl.BlockSpec((1, 3 * H), lambda i: (0, 0)),
                  pl.BlockSpec((1, 3 * H), lambda i: (0, 0)),
                  pl.BlockSpec((H, H), lambda i: (0, 0))],
        out_specs=[pl.BlockSpec((MBLK, H), lambda i: (i, 0)),
                   pl.BlockSpec((MBLK, HH), lambda i: (i, 0)),
                   pl.BlockSpec((MBLK, HH), lambda i: (i, 0))],
        out_shape=[jax.ShapeDtypeStruct((N, H), F32),
                   jax.ShapeDtypeStruct((N, HH), F32),
                   jax.ShapeDtypeStruct((N, HH), F32)],
    )(h, alo, ahi, wihT, whhT, bih, bhh, wnext)


def _tc_gru_body(h_ref, alo_ref, ahi_ref, wihT_ref, whhT_ref, bih_ref,
                 bhh_ref, hout_ref):
    agg = jnp.concatenate([alo_ref[...], ahi_ref[...]], axis=1)
    hout_ref[...] = _gru_math(h_ref[...], agg, wihT_ref[...], whhT_ref[...],
                              bih_ref[...], bhh_ref[...])


def _tc_gru(h, alo, ahi, wihT, whhT, bih, bhh):
    return pl.pallas_call(
        _tc_gru_body,
        grid=(N // MBLK,),
        in_specs=[pl.BlockSpec((MBLK, H), lambda i: (i, 0)),
                  pl.BlockSpec((MBLK, HH), lambda i: (i, 0)),
                  pl.BlockSpec((MBLK, HH), lambda i: (i, 0)),
                  pl.BlockSpec((H, 3 * H), lambda i: (0, 0)),
                  pl.BlockSpec((H, 3 * H), lambda i: (0, 0)),
                  pl.BlockSpec((1, 3 * H), lambda i: (0, 0)),
                  pl.BlockSpec((1, 3 * H), lambda i: (0, 0))],
        out_specs=pl.BlockSpec((MBLK, H), lambda i: (i, 0)),
        out_shape=jax.ShapeDtypeStruct((N, H), F32),
    )(h, alo, ahi, wihT, whhT, bih, bhh)


def _tc_readout_body(h_ref, batch_ref, w1T_ref, w2T_ref, b12_ref, qw_ref,
                     qb_ref, w3aT_ref, w3bT_ref, b3_ref, sh_ref):
    h = h_ref[...]
    bvec = batch_ref[...]
    iota_b = lax.broadcasted_iota(jnp.int32, (N, B), 1)
    moh = (bvec == iota_b).astype(F32)
    csum = jnp.sum((bvec <= iota_b).astype(F32), axis=0, keepdims=True)
    lidx = csum - 1.0
    lidx = jnp.where(lidx < 0.0, lidx + float(N), lidx)
    iota_n = lax.broadcasted_iota(F32, (N, B), 0)
    psel = (iota_n == lidx).astype(F32)
    v_n = _dot_c0(psel, h)
    vrep = _dotT(moh, v_n)
    pre = _dotT(vrep, w1T_ref[...]) + _dotT(h, w2T_ref[...]) + b12_ref[...]
    sig = jax.nn.sigmoid(pre)
    alpha = jnp.sum(sig * qw_ref[...], axis=1, keepdims=True) + qb_ref[...]
    s_g = _dot_c0(moh, alpha * h)
    sh_ref[...] = (_dotT(v_n, w3aT_ref[...]) + _dotT(s_g, w3bT_ref[...])
                   + b3_ref[...])


def _tc_readout(h, batch2d, w1T, w2T, b12, qw, qb, w3aT, w3bT, b3):
    return pl.pallas_call(
        _tc_readout_body,
        out_shape=jax.ShapeDtypeStruct((B, H), F32),
    )(h, batch2d, w1T, w2T, b12, qw, qb, w3aT, w3bT, b3)


ZBLK = 8192


def _tc_logits_body(sh_ref, emb_ref, z_ref):
    z_ref[...] = lax.dot_general(
        sh_ref[...], emb_ref[...], (((1,), (1,)), ((), ())),
        preferred_element_type=F32, precision=HIGH)


def _tc_logits(s_h, emb):
    n_vocab = emb.shape[0]
    grid = (n_vocab + ZBLK - 1) // ZBLK
    return pl.pallas_call(
        _tc_logits_body,
        grid=(grid,),
        in_specs=[pl.BlockSpec((B, H), lambda i: (0, 0)),
                  pl.BlockSpec((ZBLK, H), lambda i: (i, 0))],
        out_specs=pl.BlockSpec((B, ZBLK), lambda i: (0, i)),
        out_shape=jax.ShapeDtypeStruct((B, n_vocab), F32),
    )(s_h, emb)


def kernel(x, edge_index, batch, edge_attr, emb, gg_weight, w_ih, w_hh,
           b_ih, b_hh, W1, b1, W2, b2, q_w, q_b, W3, b3):
    del edge_attr
    xm1 = (x - 1).reshape(N // GCH, GCH)
    src = edge_index[0]
    dst = edge_index[1]

    wihT = w_ih.T
    whhT = w_hh.T
    bih = b_ih.reshape(1, 3 * H)
    bhh = b_hh.reshape(1, 3 * H)

    h0 = _sc_gather(emb, xm1)

    mlo, mhi = _tc_m(h0, gg_weight[0])
    alo, ahi = _sc_edge_agg(mlo, mhi, src, dst)
    h1, mlo1, mhi1 = _tc_gru_m(h0, alo, ahi, wihT, whhT, bih, bhh,
                               gg_weight[1])
    alo1, ahi1 = _sc_edge_agg(mlo1, mhi1, src, dst)
    h2 = _tc_gru(h1, alo1, ahi1, wihT, whhT, bih, bhh)

    s_h = _tc_readout(
        h2, batch.astype(jnp.int32).reshape(N, 1),
        W1.T, W2.T, (b1 + b2).reshape(1, H),
        q_w.reshape(1, H), q_b.reshape(1, 1),
        W3[:, :H].T, W3[:, H:].T, b3.reshape(1, H))

    return _tc_logits(s_h, emb)


# speedup vs baseline: 2.4240x; 2.4240x over previous
import functools

import jax
import jax.numpy as jnp
from jax import lax
from jax.experimental import pallas as pl
from jax.experimental.pallas import tpu as pltpu
from jax.experimental.pallas import tpu_sc as plsc

H = 128
HH = H // 2
N = 16384
E = 65536
B = 16
NC, NS, LANES = 2, 16, 16
NW = NC * NS

F32 = jnp.float32
HIGH = lax.Precision.HIGHEST


def _dotT(a, b):
    return jnp.dot(a, b, preferred_element_type=F32, precision=HIGH)


def _dot_c0(a, b):
    return lax.dot_general(a, b, (((0,), (0,)), ((), ())),
                           preferred_element_type=F32, precision=HIGH)


ROWS_PER_W = N // NW
GCH = 128
GITER = ROWS_PER_W // GCH

@functools.cache
def _sc_mesh():
    return plsc.VectorSubcoreMesh(core_axis_name="c", subcore_axis_name="s",
                                  num_cores=NC, num_subcores=NS)


@functools.cache
def _sc_gather_fn():
    @functools.partial(
        pl.kernel,
        out_type=jax.ShapeDtypeStruct((N, H), F32),
        mesh=_sc_mesh(),
        scratch_types=[
            pltpu.VMEM((GITER, GCH), jnp.int32),
            pltpu.VMEM((GCH, H), F32),
            pltpu.SemaphoreType.DMA,
        ],
    )
    def _sc_gather(table_hbm, idx_hbm, out_hbm, idx_v, rows_v, sem):
        wid = lax.axis_index("s") * NC + lax.axis_index("c")
        base = wid * ROWS_PER_W
        pltpu.sync_copy(idx_hbm.at[pl.ds(wid * GITER, GITER)], idx_v)
        for j in range(GITER):
            pltpu.async_copy(table_hbm.at[idx_v.at[j]], rows_v, sem).wait()
            pltpu.sync_copy(rows_v, out_hbm.at[pl.ds(base + j * GCH, GCH)])

    return _sc_gather


NHALF = N // NC
E_PER_TILE = E // NS
ECH = 128
EITER = E_PER_TILE // ECH
OUT_ROWS_PER_TILE = NHALF // NS


@functools.cache
def _sc_edge_agg_fn():
    @functools.partial(
        pl.kernel,
        out_type=jax.ShapeDtypeStruct((N, H), F32),
        mesh=_sc_mesh(),
        scratch_types=[
            pltpu.VMEM((ECH,), jnp.int32),
            pltpu.VMEM((ECH,), jnp.int32),
            pltpu.VMEM((ECH, H), F32),
            pltpu.VMEM_SHARED((NHALF + ECH, H), F32),
            pltpu.SemaphoreType.DMA,
        ],
    )
    def _sc_edge_agg(m_hbm, src_hbm, dst_hbm, out_hbm,
                     srcv, dstv, rows_v, agg_sh, sem):
        c = lax.axis_index("c")
        s = lax.axis_index("s")

        def _zrow(i, carry):
            for q in range(H // LANES):
                rows_v[i, pl.ds(q * LANES, LANES)] = jnp.zeros((LANES,), F32)
            return carry
        lax.fori_loop(0, ECH, _zrow, 0)
        for j in range(OUT_ROWS_PER_TILE // ECH):
            pltpu.sync_copy(
                rows_v,
                agg_sh.at[pl.ds(s * OUT_ROWS_PER_TILE + j * ECH, ECH)])

        @pl.when(s == 0)
        def _():
            pltpu.sync_copy(rows_v, agg_sh.at[pl.ds(NHALF, ECH)])

        plsc.subcore_barrier()

        rbase = c * NHALF

        def _edge_chunk(i, carry):
            off = s * E_PER_TILE + i * ECH
            pltpu.sync_copy(src_hbm.at[pl.ds(off, ECH)], srcv)
            pltpu.sync_copy(dst_hbm.at[pl.ds(off, ECH)], dstv)
            for q in range(ECH // LANES):
                d = dstv[pl.ds(q * LANES, LANES)] - rbase
                bad = (d < 0) | (d >= NHALF)
                dstv[pl.ds(q * LANES, LANES)] = jnp.where(bad, NHALF, d)
            pltpu.async_copy(m_hbm.at[srcv], rows_v, sem).wait()
            pltpu.sync_copy(rows_v, agg_sh.at[dstv], add=True)
            return carry
        lax.fori_loop(0, EITER, _edge_chunk, 0)
        plsc.subcore_barrier()

        pltpu.sync_copy(
            agg_sh.at[pl.ds(s * OUT_ROWS_PER_TILE, OUT_ROWS_PER_TILE)],
            out_hbm.at[pl.ds(rbase + s * OUT_ROWS_PER_TILE,
                             OUT_ROWS_PER_TILE)])

    return _sc_edge_agg


MBLK = 2048


def _tc_m_body(h_ref, w_ref, m_ref):
    m_ref[...] = _dotT(h_ref[...], w_ref[...])


def _tc_m(h, w):
    return pl.pallas_call(
        _tc_m_body,
        grid=(N // MBLK,),
        in_specs=[pl.BlockSpec((MBLK, H), lambda i: (i, 0)),
                  pl.BlockSpec((H, H), lambda i: (0, 0))],
        out_specs=pl.BlockSpec((MBLK, H), lambda i: (i, 0)),
        out_shape=jax.ShapeDtypeStruct((N, H), F32),
    )(h, w)


def _gru_math(h, agg, wihT, whhT, bih, bhh):
    gi = _dotT(agg, wihT) + bih
    gh = _dotT(h, whhT) + bhh
    i_r, i_z, i_n = gi[:, :H], gi[:, H:2 * H], gi[:, 2 * H:]
    h_r, h_z, h_n = gh[:, :H], gh[:, H:2 * H], gh[:, 2 * H:]
    r = jax.nn.sigmoid(i_r + h_r)
    z = jax.nn.sigmoid(i_z + h_z)
    n = jnp.tanh(i_n + r * h_n)
    return (1.0 - z) * n + z * h


def _tc_gru_m_body(h_ref, agg_ref, wihT_ref, whhT_ref, bih_ref,
                   bhh_ref, wnext_ref, hout_ref, m_ref):
    hn = _gru_math(h_ref[...], agg_ref[...], wihT_ref[...], whhT_ref[...],
                   bih_ref[...], bhh_ref[...])
    hout_ref[...] = hn
    m_ref[...] = _dotT(hn, wnext_ref[...])


def _tc_gru_m(h, agg, wihT, whhT, bih, bhh, wnext):
    return pl.pallas_call(
        _tc_gru_m_body,
        grid=(N // MBLK,),
        in_specs=[pl.BlockSpec((MBLK, H), lambda i: (i, 0)),
                  pl.BlockSpec((MBLK, H), lambda i: (i, 0)),
                  pl.BlockSpec((H, 3 * H), lambda i: (0, 0)),
                  pl.BlockSpec((H, 3 * H), lambda i: (0, 0)),
                  pl.BlockSpec((1, 3 * H), lambda i: (0, 0)),
                  pl.BlockSpec((1, 3 * H), lambda i: (0, 0)),
                  pl.BlockSpec((H, H), lambda i: (0, 0))],
        out_specs=[pl.BlockSpec((MBLK, H), lambda i: (i, 0)),
                   pl.BlockSpec((MBLK, H), lambda i: (i, 0))],
        out_shape=[jax.ShapeDtypeStruct((N, H), F32),
                   jax.ShapeDtypeStruct((N, H), F32)],
    )(h, agg, wihT, whhT, bih, bhh, wnext)


def _tc_gru_body(h_ref, agg_ref, wihT_ref, whhT_ref, bih_ref,
                 bhh_ref, hout_ref):
    hout_ref[...] = _gru_math(h_ref[...], agg_ref[...], wihT_ref[...],
                              whhT_ref[...], bih_ref[...], bhh_ref[...])


def _tc_gru(h, agg, wihT, whhT, bih, bhh):
    return pl.pallas_call(
        _tc_gru_body,
        grid=(N // MBLK,),
        in_specs=[pl.BlockSpec((MBLK, H), lambda i: (i, 0)),
                  pl.BlockSpec((MBLK, H), lambda i: (i, 0)),
                  pl.BlockSpec((H, 3 * H), lambda i: (0, 0)),
                  pl.BlockSpec((H, 3 * H), lambda i: (0, 0)),
                  pl.BlockSpec((1, 3 * H), lambda i: (0, 0)),
                  pl.BlockSpec((1, 3 * H), lambda i: (0, 0))],
        out_specs=pl.BlockSpec((MBLK, H), lambda i: (i, 0)),
        out_shape=jax.ShapeDtypeStruct((N, H), F32),
    )(h, agg, wihT, whhT, bih, bhh)


RBLK = 2048


def _tc_vn_body(h_ref, b_ref, bn_ref, v1_ref, cnt_ref):
    i = pl.program_id(0)
    bvec = b_ref[...]
    iota_b = lax.broadcasted_iota(jnp.int32, (RBLK, B), 1)
    moh = (bvec == iota_b).astype(F32)
    islast = (bvec != bn_ref[...]).astype(F32)
    v1 = _dot_c0(moh * islast, h_ref[...])
    cnt = jnp.broadcast_to(jnp.sum(moh, axis=0, keepdims=True), (8, B))

    @pl.when(i == 0)
    def _():
        v1_ref[...] = v1
        cnt_ref[...] = cnt

    @pl.when(i > 0)
    def _():
        v1_ref[...] += v1
        cnt_ref[...] += cnt


def _tc_vn(h, batch2d, batchnext2d):
    return pl.pallas_call(
        _tc_vn_body,
        grid=(N // RBLK,),
        in_specs=[pl.BlockSpec((RBLK, H), lambda i: (i, 0)),
                  pl.BlockSpec((RBLK, 1), lambda i: (i, 0)),
                  pl.BlockSpec((RBLK, 1), lambda i: (i, 0))],
        out_specs=[pl.BlockSpec((B, H), lambda i: (0, 0)),
                   pl.BlockSpec((8, B), lambda i: (0, 0))],
        out_shape=[jax.ShapeDtypeStruct((B, H), F32),
                   jax.ShapeDtypeStruct((8, B), F32)],
    )(h, batch2d, batchnext2d)


def _fixup_vn(v1, cnt_row, hlast):
    ii = lax.broadcasted_iota(jnp.int32, (B, B), 0)
    bb = lax.broadcasted_iota(jnp.int32, (B, B), 1)
    eye = (ii == bb).astype(F32)
    cand = jnp.where(
        cnt_row > 0.0,
        lax.broadcasted_iota(jnp.int32, (1, B), 1).astype(F32), -1.0)
    candT = lax.dot_general(eye, cand, (((1,), (1,)), ((), ())),
                            preferred_element_type=F32,
                            precision=HIGH)
    jmat = jnp.where(ii <= bb, jnp.broadcast_to(candT, (B, B)), -1.0)
    jrow = jnp.max(jmat, axis=0, keepdims=True)
    onehot = (ii == jrow.astype(jnp.int32)).astype(F32)
    v_n = _dot_c0(onehot, v1)
    jT = lax.dot_general(eye, jrow, (((1,), (1,)), ((), ())),
                         preferred_element_type=F32, precision=HIGH)
    return jnp.where(jT < 0.0, jnp.broadcast_to(hlast, (B, H)), v_n)


def _tc_readout_body(h_ref, b_ref, v1_ref, cnt_ref, hlast_ref, w1T_ref,
                     w2T_ref, b12_ref, qw_ref, qb_ref, w3aT_ref, w3bT_ref,
                     b3_ref, sh_ref):
    i = pl.program_id(0)
    v_n = _fixup_vn(v1_ref[...], cnt_ref[0:1, :], hlast_ref[...])
    h = h_ref[...]
    bvec = b_ref[...]
    iota_b = lax.broadcasted_iota(jnp.int32, (RBLK, B), 1)
    moh = (bvec == iota_b).astype(F32)
    vrep = _dotT(moh, v_n)
    pre = _dotT(vrep, w1T_ref[...]) + _dotT(h, w2T_ref[...]) + b12_ref[...]
    sig = jax.nn.sigmoid(pre)
    alpha = (jnp.sum(sig * qw_ref[...], axis=1, keepdims=True)
             + qb_ref[...][:, :1])
    s_g = _dot_c0(moh, alpha * h)
    part = _dotT(s_g, w3bT_ref[...])

    @pl.when(i == 0)
    def _():
        sh_ref[...] = _dotT(v_n, w3aT_ref[...]) + b3_ref[...] + part

    @pl.when(i > 0)
    def _():
        sh_ref[...] += part


def _tc_readout(h, batch2d, v1, cnt, hlast, w1T, w2T, b12, qw, qb,
                w3aT, w3bT, b3):
    zero = lambda i: (0, 0)
    return pl.pallas_call(
        _tc_readout_body,
        grid=(N // RBLK,),
        in_specs=[pl.BlockSpec((RBLK, H), lambda i: (i, 0)),
                  pl.BlockSpec((RBLK, 1), lambda i: (i, 0)),
                  pl.BlockSpec((B, H), zero),
                  pl.BlockSpec((8, B), zero),
                  pl.BlockSpec((1, H), zero),
                  pl.BlockSpec((H, H), zero),
                  pl.BlockSpec((H, H), zero),
                  pl.BlockSpec((1, H), zero),
                  pl.BlockSpec((1, H), zero),
                  pl.BlockSpec((1, H), zero),
                  pl.BlockSpec((H, H), zero),
                  pl.BlockSpec((H, H), zero),
                  pl.BlockSpec((1, H), zero)],
        out_specs=pl.BlockSpec((B, H), zero),
        out_shape=jax.ShapeDtypeStruct((B, H), F32),
    )(h, batch2d, v1, cnt, hlast, w1T, w2T, b12, qw, qb, w3aT, w3bT, b3)


ZBLK = 8192


def _tc_logits_body(sh_ref, emb_ref, z_ref):
    z_ref[...] = lax.dot_general(
        sh_ref[...], emb_ref[...], (((1,), (1,)), ((), ())),
        preferred_element_type=F32, precision=HIGH)


def _tc_logits(s_h, emb):
    n_vocab = emb.shape[0]
    grid = (n_vocab + ZBLK - 1) // ZBLK
    return pl.pallas_call(
        _tc_logits_body,
        grid=(grid,),
        in_specs=[pl.BlockSpec((B, H), lambda i: (0, 0)),
                  pl.BlockSpec((ZBLK, H), lambda i: (i, 0))],
        out_specs=pl.BlockSpec((B, ZBLK), lambda i: (0, i)),
        out_shape=jax.ShapeDtypeStruct((B, n_vocab), F32),
    )(s_h, emb)


def kernel(x, edge_index, batch, edge_attr, emb, gg_weight, w_ih, w_hh,
           b_ih, b_hh, W1, b1, W2, b2, q_w, q_b, W3, b3):
    del edge_attr
    xm1 = (x - 1).reshape(N // GCH, GCH)
    src = edge_index[0]
    dst = edge_index[1]

    wihT = w_ih.T
    whhT = w_hh.T
    bih = b_ih.reshape(1, 3 * H)
    bhh = b_hh.reshape(1, 3 * H)

    h0 = _sc_gather_fn()(emb, xm1)

    m0 = _tc_m(h0, gg_weight[0])
    agg0 = _sc_edge_agg_fn()(m0, src, dst)
    h1, m1 = _tc_gru_m(h0, agg0, wihT, whhT, bih, bhh, gg_weight[1])
    agg1 = _sc_edge_agg_fn()(m1, src, dst)
    h2 = _tc_gru(h1, agg1, wihT, whhT, bih, bhh)

    batch2d = batch.astype(jnp.int32).reshape(N, 1)
    batchnext2d = jnp.concatenate(
        [batch2d[1:], jnp.full((1, 1), -1, jnp.int32)], axis=0)
    v1, cnt = _tc_vn(h2, batch2d, batchnext2d)
    s_h = _tc_readout(
        h2, batch2d, v1, cnt, h2[N - 1:N],
        W1.T, W2.T, (b1 + b2).reshape(1, H),
        q_w.reshape(1, H), jnp.broadcast_to(q_b.reshape(1, 1), (1, H)),
        W3[:, :H].T, W3[:, H:].T, b3.reshape(1, H))

    return _tc_logits(s_h, emb)


# speedup vs baseline: 4.5909x; 1.8939x over previous
import functools

import jax
import jax.numpy as jnp
from jax import lax
from jax.experimental import pallas as pl
from jax.experimental.pallas import tpu as pltpu
from jax.experimental.pallas import tpu_sc as plsc

H = 128
N = 16384
E = 65536
B = 16
NC, NS, LANES = 2, 16, 16
NW = NC * NS

F32 = jnp.float32
HIGH = lax.Precision.HIGHEST


def _dotT(a, b):
    return jnp.dot(a, b, preferred_element_type=F32, precision=HIGH)


def _dotD(a, b):
    return jnp.dot(a, b, preferred_element_type=F32)


def _dot_c0(a, b):
    return lax.dot_general(a, b, (((0,), (0,)), ((), ())),
                           preferred_element_type=F32, precision=HIGH)


ROWS_PER_W = N // NW
GCH = 128
GITER = ROWS_PER_W // GCH

_SC_MESH_KW = dict(core_axis_name="c", subcore_axis_name="s",
                   num_cores=NC, num_subcores=NS)


@functools.cache
def _sc_mesh():
    return plsc.VectorSubcoreMesh(**_SC_MESH_KW)


@functools.cache
def _sc_gather_fn():
    @functools.partial(
        pl.kernel,
        out_type=jax.ShapeDtypeStruct((N, H), F32),
        mesh=_sc_mesh(),
        scratch_types=[
            pltpu.VMEM((GITER, GCH), jnp.int32),
            pltpu.VMEM((GCH, H), F32),
            pltpu.VMEM((GCH, H), F32),
            pltpu.SemaphoreType.DMA,
            pltpu.SemaphoreType.DMA,
        ],
    )
    def _sc_gather(table_hbm, idx_hbm, out_hbm, idx_v, rows0, rows1,
                   sem0, sem1):
        wid = lax.axis_index("s") * NC + lax.axis_index("c")
        base = wid * ROWS_PER_W
        pltpu.sync_copy(idx_hbm.at[pl.ds(wid * GITER, GITER)], idx_v)
        bufs = (rows0, rows1)
        sems = (sem0, sem1)
        pltpu.async_copy(table_hbm.at[idx_v.at[0]], rows0, sem0)
        for j in range(GITER):
            b = j % 2
            if j + 1 < GITER:
                pltpu.async_copy(table_hbm.at[idx_v.at[j + 1]],
                                 bufs[1 - b], sems[1 - b])
            pltpu.make_async_copy(table_hbm.at[idx_v.at[j]],
                                  bufs[b], sems[b]).wait()
            pltpu.sync_copy(bufs[b], out_hbm.at[pl.ds(base + j * GCH, GCH)])

    return _sc_gather


NHALF = N // NC
E_PER_TILE = E // NS
ECH = 128
EITER = E_PER_TILE // ECH
OUT_ROWS_PER_TILE = NHALF // NS


@functools.cache
def _sc_edge_agg_fn():
    @functools.partial(
        pl.kernel,
        out_type=jax.ShapeDtypeStruct((N, H), F32),
        mesh=_sc_mesh(),
        scratch_types=[
            pltpu.VMEM((EITER, ECH), jnp.int32),
            pltpu.VMEM((EITER, ECH), jnp.int32),
            pltpu.VMEM((ECH, H), F32),
            pltpu.VMEM((ECH, H), F32),
            pltpu.VMEM_SHARED((NHALF + ECH, H), F32),
            pltpu.SemaphoreType.DMA,
            pltpu.SemaphoreType.DMA,
        ],
    )
    def _sc_edge_agg(h_hbm, src_hbm, dst_hbm, out_hbm,
                     srcv, dstv, rows0, rows1, agg_sh, sem0, sem1):
        c = lax.axis_index("c")
        s = lax.axis_index("s")

        pltpu.sync_copy(src_hbm.at[pl.ds(s * EITER, EITER)], srcv)
        pltpu.sync_copy(dst_hbm.at[pl.ds(s * EITER, EITER)], dstv)

        rbase = c * NHALF

        def _locrow(j, carry):
            for q in range(ECH // LANES):
                d = dstv[j, pl.ds(q * LANES, LANES)] - rbase
                bad = (d < 0) | (d >= NHALF)
                dstv[j, pl.ds(q * LANES, LANES)] = jnp.where(bad, NHALF, d)
            return carry
        lax.fori_loop(0, EITER, _locrow, 0)

        def _zrow(i, carry):
            for q in range(H // LANES):
                rows0[i, pl.ds(q * LANES, LANES)] = jnp.zeros((LANES,), F32)
            return carry
        lax.fori_loop(0, ECH, _zrow, 0)
        for j in range(OUT_ROWS_PER_TILE // ECH):
            pltpu.sync_copy(
                rows0,
                agg_sh.at[pl.ds(s * OUT_ROWS_PER_TILE + j * ECH, ECH)])

        @pl.when(s == 0)
        def _():
            pltpu.sync_copy(rows0, agg_sh.at[pl.ds(NHALF, ECH)])

        plsc.subcore_barrier()

        bufs = (rows0, rows1)
        sems = (sem0, sem1)
        pltpu.async_copy(h_hbm.at[srcv.at[0]], rows0, sem0)
        for j in range(EITER):
            b = j % 2
            if j + 1 < EITER:
                pltpu.async_copy(h_hbm.at[srcv.at[j + 1]],
                                 bufs[1 - b], sems[1 - b])
            pltpu.make_async_copy(h_hbm.at[srcv.at[j]],
                                  bufs[b], sems[b]).wait()
            pltpu.sync_copy(bufs[b], agg_sh.at[dstv.at[j]], add=True)
        plsc.subcore_barrier()

        pltpu.sync_copy(
            agg_sh.at[pl.ds(s * OUT_ROWS_PER_TILE, OUT_ROWS_PER_TILE)],
            out_hbm.at[pl.ds(rbase + s * OUT_ROWS_PER_TILE,
                             OUT_ROWS_PER_TILE)])

    return _sc_edge_agg


MBLK = 2048


def _gru_math(h, aggh, gg_ref, wihT_ref, whhT_ref, bih_ref, bhh_ref):
    wA = _dotD(gg_ref[...], wihT_ref[...])
    gi = _dotD(aggh, wA) + bih_ref[...]
    gh = _dotD(h, whhT_ref[...]) + bhh_ref[...]
    i_r, i_z, i_n = gi[:, :H], gi[:, H:2 * H], gi[:, 2 * H:]
    h_r, h_z, h_n = gh[:, :H], gh[:, H:2 * H], gh[:, 2 * H:]
    r = jax.nn.sigmoid(i_r + h_r)
    z = jax.nn.sigmoid(i_z + h_z)
    n = jnp.tanh(i_n + r * h_n)
    return (1.0 - z) * n + z * h


def _tc_gru_body(h_ref, agg_ref, gg_ref, wihT_ref, whhT_ref, bih_ref,
                 bhh_ref, hout_ref):
    hout_ref[...] = _gru_math(h_ref[...], agg_ref[...], gg_ref, wihT_ref,
                              whhT_ref, bih_ref, bhh_ref)


def _tc_gru(h, aggh, gg, wihT, whhT, bih, bhh):
    zero = lambda i: (0, 0)
    return pl.pallas_call(
        _tc_gru_body,
        grid=(N // MBLK,),
        in_specs=[pl.BlockSpec((MBLK, H), lambda i: (i, 0)),
                  pl.BlockSpec((MBLK, H), lambda i: (i, 0)),
                  pl.BlockSpec((H, H), zero),
                  pl.BlockSpec((H, 3 * H), zero),
                  pl.BlockSpec((H, 3 * H), zero),
                  pl.BlockSpec((1, 3 * H), zero),
                  pl.BlockSpec((1, 3 * H), zero)],
        out_specs=pl.BlockSpec((MBLK, H), lambda i: (i, 0)),
        out_shape=jax.ShapeDtypeStruct((N, H), F32),
    )(h, aggh, gg, wihT, whhT, bih, bhh)


def _tc_gru_vn_body(h_ref, agg_ref, gg_ref, wihT_ref, whhT_ref, bih_ref,
                    bhh_ref, b_ref, bn_ref, hout_ref, v1_ref, cnt_ref):
    i = pl.program_id(0)
    hn = _gru_math(h_ref[...], agg_ref[...], gg_ref, wihT_ref,
                   whhT_ref, bih_ref, bhh_ref)
    hout_ref[...] = hn
    bvec = b_ref[...]
    iota_b = lax.broadcasted_iota(jnp.int32, (MBLK, B), 1)
    moh = (bvec == iota_b).astype(F32)
    islast = (bvec != bn_ref[...]).astype(F32)
    v1 = _dot_c0(moh * islast, hn)
    cnt = jnp.broadcast_to(jnp.sum(moh, axis=0, keepdims=True), (8, B))

    @pl.when(i == 0)
    def _():
        v1_ref[...] = v1
        cnt_ref[...] = cnt

    @pl.when(i > 0)
    def _():
        v1_ref[...] += v1
        cnt_ref[...] += cnt


def _tc_gru_vn(h, aggh, gg, wihT, whhT, bih, bhh, batch2d, batchnext2d):
    zero = lambda i: (0, 0)
    return pl.pallas_call(
        _tc_gru_vn_body,
        grid=(N // MBLK,),
        in_specs=[pl.BlockSpec((MBLK, H), lambda i: (i, 0)),
                  pl.BlockSpec((MBLK, H), lambda i: (i, 0)),
                  pl.BlockSpec((H, H), zero),
                  pl.BlockSpec((H, 3 * H), zero),
                  pl.BlockSpec((H, 3 * H), zero),
                  pl.BlockSpec((1, 3 * H), zero),
                  pl.BlockSpec((1, 3 * H), zero),
                  pl.BlockSpec((MBLK, 1), lambda i: (i, 0)),
                  pl.BlockSpec((MBLK, 1), lambda i: (i, 0))],
        out_specs=[pl.BlockSpec((MBLK, H), lambda i: (i, 0)),
                   pl.BlockSpec((B, H), zero),
                   pl.BlockSpec((8, B), zero)],
        out_shape=[jax.ShapeDtypeStruct((N, H), F32),
                   jax.ShapeDtypeStruct((B, H), F32),
                   jax.ShapeDtypeStruct((8, B), F32)],
    )(h, aggh, gg, wihT, whhT, bih, bhh, batch2d, batchnext2d)


RBLK = 2048


def _fixup_vn(v1, cnt_row, hlast):
    ii = lax.broadcasted_iota(jnp.int32, (B, B), 0)
    bb = lax.broadcasted_iota(jnp.int32, (B, B), 1)
    eye = (ii == bb).astype(F32)
    cand = jnp.where(
        cnt_row > 0.0,
        lax.broadcasted_iota(jnp.int32, (1, B), 1).astype(F32), -1.0)
    candT = lax.dot_general(eye, cand, (((1,), (1,)), ((), ())),
                            preferred_element_type=F32,
                            precision=HIGH)
    jmat = jnp.where(ii <= bb, jnp.broadcast_to(candT, (B, B)), -1.0)
    jrow = jnp.max(jmat, axis=0, keepdims=True)
    onehot = (ii == jrow.astype(jnp.int32)).astype(F32)
    v_n = _dot_c0(onehot, v1)
    jT = lax.dot_general(eye, jrow, (((1,), (1,)), ((), ())),
                         preferred_element_type=F32, precision=HIGH)
    return jnp.where(jT < 0.0, jnp.broadcast_to(hlast, (B, H)), v_n)


def _tc_readout_body(h_ref, b_ref, v1_ref, cnt_ref, hlast_ref, w1T_ref,
                     w2T_ref, b12_ref, qw_ref, qb_ref, w3aT_ref, w3bT_ref,
                     b3_ref, sh_ref):
    i = pl.program_id(0)
    v_n = _fixup_vn(v1_ref[...], cnt_ref[0:1, :], hlast_ref[...])
    h = h_ref[...]
    bvec = b_ref[...]
    iota_b = lax.broadcasted_iota(jnp.int32, (RBLK, B), 1)
    moh = (bvec == iota_b).astype(F32)
    vrep = _dotT(moh, v_n)
    pre = _dotD(vrep, w1T_ref[...]) + _dotD(h, w2T_ref[...]) + b12_ref[...]
    sig = jax.nn.sigmoid(pre)
    alpha = (jnp.sum(sig * qw_ref[...], axis=1, keepdims=True)
             + qb_ref[...][:, :1])
    s_g = _dot_c0(moh, alpha * h)
    part = _dotD(s_g, w3bT_ref[...])

    @pl.when(i == 0)
    def _():
        sh_ref[...] = _dotD(v_n, w3aT_ref[...]) + b3_ref[...] + part

    @pl.when(i > 0)
    def _():
        sh_ref[...] += part


def _tc_readout(h, batch2d, v1, cnt, hlast, w1T, w2T, b12, qw, qb,
                w3aT, w3bT, b3):
    zero = lambda i: (0, 0)
    return pl.pallas_call(
        _tc_readout_body,
        grid=(N // RBLK,),
        in_specs=[pl.BlockSpec((RBLK, H), lambda i: (i, 0)),
                  pl.BlockSpec((RBLK, 1), lambda i: (i, 0)),
                  pl.BlockSpec((B, H), zero),
                  pl.BlockSpec((8, B), zero),
                  pl.BlockSpec((1, H), zero),
                  pl.BlockSpec((H, H), zero),
                  pl.BlockSpec((H, H), zero),
                  pl.BlockSpec((1, H), zero),
                  pl.BlockSpec((1, H), zero),
                  pl.BlockSpec((1, H), zero),
                  pl.BlockSpec((H, H), zero),
                  pl.BlockSpec((H, H), zero),
                  pl.BlockSpec((1, H), zero)],
        out_specs=pl.BlockSpec((B, H), zero),
        out_shape=jax.ShapeDtypeStruct((B, H), F32),
    )(h, batch2d, v1, cnt, hlast, w1T, w2T, b12, qw, qb, w3aT, w3bT, b3)


ZBLK = 16384


def _tc_logits_body(sh_ref, emb_ref, z_ref):
    z_ref[...] = lax.dot_general(
        sh_ref[...], emb_ref[...], (((1,), (1,)), ((), ())),
        preferred_element_type=F32)


def _tc_logits(s_h, emb):
    n_vocab = emb.shape[0]
    grid = (n_vocab + ZBLK - 1) // ZBLK
    return pl.pallas_call(
        _tc_logits_body,
        grid=(grid,),
        in_specs=[pl.BlockSpec((B, H), lambda i: (0, 0)),
                  pl.BlockSpec((ZBLK, H), lambda i: (i, 0))],
        out_specs=pl.BlockSpec((B, ZBLK), lambda i: (0, i)),
        out_shape=jax.ShapeDtypeStruct((B, n_vocab), F32),
    )(s_h, emb)


def kernel(x, edge_index, batch, edge_attr, emb, gg_weight, w_ih, w_hh,
           b_ih, b_hh, W1, b1, W2, b2, q_w, q_b, W3, b3):
    del edge_attr
    xm1 = (x - 1).reshape(N // GCH, GCH)
    src2d = edge_index[0].reshape(E // ECH, ECH)
    dst2d = edge_index[1].reshape(E // ECH, ECH)

    wihT = w_ih.T
    whhT = w_hh.T
    bih = b_ih.reshape(1, 3 * H)
    bhh = b_hh.reshape(1, 3 * H)

    h0 = _sc_gather_fn()(emb, xm1)

    batch2d = batch.astype(jnp.int32).reshape(N, 1)
    batchnext2d = jnp.concatenate(
        [batch2d[1:], jnp.full((1, 1), -1, jnp.int32)], axis=0)

    aggh0 = _sc_edge_agg_fn()(h0, src2d, dst2d)
    h1 = _tc_gru(h0, aggh0, gg_weight[0], wihT, whhT, bih, bhh)
    aggh1 = _sc_edge_agg_fn()(h1, src2d, dst2d)
    h2, v1, cnt = _tc_gru_vn(h1, aggh1, gg_weight[1], wihT, whhT, bih, bhh,
                             batch2d, batchnext2d)

    s_h = _tc_readout(
        h2, batch2d, v1, cnt, h2[N - 1:N],
        W1.T, W2.T, (b1 + b2).reshape(1, H),
        q_w.reshape(1, H), jnp.broadcast_to(q_b.reshape(1, 1), (1, H)),
        W3[:, :H].T, W3[:, H:].T, b3.reshape(1, H))

    return _tc_logits(s_h, emb)


# speedup vs baseline: 4.6010x; 1.0022x over previous
import functools

import jax
import jax.numpy as jnp
from jax import lax
from jax.experimental import pallas as pl
from jax.experimental.pallas import tpu as pltpu
from jax.experimental.pallas import tpu_sc as plsc

H = 128
N = 16384
E = 65536
B = 16
NC, NS, LANES = 2, 16, 16
NW = NC * NS

F32 = jnp.float32
HIGH = lax.Precision.HIGHEST


def _dotT(a, b):
    return jnp.dot(a, b, preferred_element_type=F32, precision=HIGH)


def _dotD(a, b):
    return jnp.dot(a, b, preferred_element_type=F32)


def _dot_c0(a, b):
    return lax.dot_general(a, b, (((0,), (0,)), ((), ())),
                           preferred_element_type=F32, precision=HIGH)


ROWS_PER_W = N // NW
GCH = 128
GITER = ROWS_PER_W // GCH

_SC_MESH_KW = dict(core_axis_name="c", subcore_axis_name="s",
                   num_cores=NC, num_subcores=NS)


@functools.cache
def _sc_mesh():
    return plsc.VectorSubcoreMesh(**_SC_MESH_KW)


@functools.cache
def _sc_gather_fn():
    @functools.partial(
        pl.kernel,
        out_type=jax.ShapeDtypeStruct((N, H), F32),
        mesh=_sc_mesh(),
        scratch_types=[
            pltpu.VMEM((GITER, GCH), jnp.int32),
            pltpu.VMEM((GCH, H), F32),
            pltpu.VMEM((GCH, H), F32),
            pltpu.SemaphoreType.DMA,
            pltpu.SemaphoreType.DMA,
        ],
    )
    def _sc_gather(table_hbm, idx_hbm, out_hbm, idx_v, rows0, rows1,
                   sem0, sem1):
        wid = lax.axis_index("s") * NC + lax.axis_index("c")
        base = wid * ROWS_PER_W
        pltpu.sync_copy(idx_hbm.at[pl.ds(wid * GITER, GITER)], idx_v)
        bufs = (rows0, rows1)
        sems = (sem0, sem1)
        pltpu.async_copy(table_hbm.at[idx_v.at[0]], rows0, sem0)
        for j in range(GITER):
            b = j % 2
            if j + 1 < GITER:
                pltpu.async_copy(table_hbm.at[idx_v.at[j + 1]],
                                 bufs[1 - b], sems[1 - b])
            pltpu.make_async_copy(table_hbm.at[idx_v.at[j]],
                                  bufs[b], sems[b]).wait()
            pltpu.sync_copy(bufs[b], out_hbm.at[pl.ds(base + j * GCH, GCH)])

    return _sc_gather


NHALF = N // NC
E_PER_TILE = E // NS
ECH = 128
EITER = E_PER_TILE // ECH
OUT_ROWS_PER_TILE = NHALF // NS


@functools.cache
def _sc_edge_agg_fn():
    @functools.partial(
        pl.kernel,
        out_type=jax.ShapeDtypeStruct((N, H), F32),
        mesh=_sc_mesh(),
        scratch_types=[
            pltpu.VMEM((EITER, ECH), jnp.int32),
            pltpu.VMEM((EITER, ECH), jnp.int32),
            pltpu.VMEM((ECH, H), F32),
            pltpu.VMEM((ECH, H), F32),
            pltpu.VMEM_SHARED((NHALF + ECH, H), F32),
            pltpu.SemaphoreType.DMA,
            pltpu.SemaphoreType.DMA,
        ],
    )
    def _sc_edge_agg(h_hbm, src_hbm, dst_hbm, out_hbm,
                     srcv, dstv, rows0, rows1, agg_sh, sem0, sem1):
        c = lax.axis_index("c")
        s = lax.axis_index("s")

        pltpu.sync_copy(src_hbm.at[pl.ds(s * EITER, EITER)], srcv)
        pltpu.sync_copy(dst_hbm.at[pl.ds(s * EITER, EITER)], dstv)

        rbase = c * NHALF

        def _locrow(j, carry):
            for q in range(ECH // LANES):
                d = dstv[j, pl.ds(q * LANES, LANES)] - rbase
                bad = (d < 0) | (d >= NHALF)
                dstv[j, pl.ds(q * LANES, LANES)] = jnp.where(bad, NHALF, d)
            return carry
        lax.fori_loop(0, EITER, _locrow, 0)

        def _zrow(i, carry):
            for q in range(H // LANES):
                rows0[i, pl.ds(q * LANES, LANES)] = jnp.zeros((LANES,), F32)
            return carry
        lax.fori_loop(0, ECH, _zrow, 0)
        for j in range(OUT_ROWS_PER_TILE // ECH):
            pltpu.sync_copy(
                rows0,
                agg_sh.at[pl.ds(s * OUT_ROWS_PER_TILE + j * ECH, ECH)])

        @pl.when(s == 0)
        def _():
            pltpu.sync_copy(rows0, agg_sh.at[pl.ds(NHALF, ECH)])

        plsc.subcore_barrier()

        bufs = (rows0, rows1)
        sems = (sem0, sem1)
        pltpu.async_copy(h_hbm.at[srcv.at[0]], rows0, sem0)
        for j in range(EITER):
            b = j % 2
            if j + 1 < EITER:
                pltpu.async_copy(h_hbm.at[srcv.at[j + 1]],
                                 bufs[1 - b], sems[1 - b])
            pltpu.make_async_copy(h_hbm.at[srcv.at[j]],
                                  bufs[b], sems[b]).wait()
            pltpu.sync_copy(bufs[b], agg_sh.at[dstv.at[j]], add=True)
        plsc.subcore_barrier()

        pltpu.sync_copy(
            agg_sh.at[pl.ds(s * OUT_ROWS_PER_TILE, OUT_ROWS_PER_TILE)],
            out_hbm.at[pl.ds(rbase + s * OUT_ROWS_PER_TILE,
                             OUT_ROWS_PER_TILE)])

    return _sc_edge_agg


MBLK = 2048


def _gru_math(h, aggh, gg_ref, wihT_ref, whhT_ref, bih_ref, bhh_ref):
    wA = _dotD(gg_ref[...], wihT_ref[...])
    gi = _dotD(aggh, wA) + bih_ref[...]
    gh = _dotD(h, whhT_ref[...]) + bhh_ref[...]
    i_r, i_z, i_n = gi[:, :H], gi[:, H:2 * H], gi[:, 2 * H:]
    h_r, h_z, h_n = gh[:, :H], gh[:, H:2 * H], gh[:, 2 * H:]
    r = jax.nn.sigmoid(i_r + h_r)
    z = jax.nn.sigmoid(i_z + h_z)
    n = jnp.tanh(i_n + r * h_n)
    return (1.0 - z) * n + z * h


def _tc_gru_body(h_ref, agg_ref, gg_ref, wihT_ref, whhT_ref, bih_ref,
                 bhh_ref, hout_ref):
    hout_ref[...] = _gru_math(h_ref[...], agg_ref[...], gg_ref, wihT_ref,
                              whhT_ref, bih_ref, bhh_ref)


def _tc_gru(h, aggh, gg, wihT, whhT, bih, bhh):
    zero = lambda i: (0, 0)
    return pl.pallas_call(
        _tc_gru_body,
        grid=(N // MBLK,),
        in_specs=[pl.BlockSpec((MBLK, H), lambda i: (i, 0)),
                  pl.BlockSpec((MBLK, H), lambda i: (i, 0)),
                  pl.BlockSpec((H, H), zero),
                  pl.BlockSpec((H, 3 * H), zero),
                  pl.BlockSpec((H, 3 * H), zero),
                  pl.BlockSpec((1, 3 * H), zero),
                  pl.BlockSpec((1, 3 * H), zero)],
        out_specs=pl.BlockSpec((MBLK, H), lambda i: (i, 0)),
        out_shape=jax.ShapeDtypeStruct((N, H), F32),
    )(h, aggh, gg, wihT, whhT, bih, bhh)


def _tc_gru_vn_body(h_ref, agg_ref, gg_ref, wihT_ref, whhT_ref, bih_ref,
                    bhh_ref, b_ref, bn_ref, hout_ref, v1_ref, cnt_ref):
    i = pl.program_id(0)
    hn = _gru_math(h_ref[...], agg_ref[...], gg_ref, wihT_ref,
                   whhT_ref, bih_ref, bhh_ref)
    hout_ref[...] = hn
    bvec = b_ref[...]
    iota_b = lax.broadcasted_iota(jnp.int32, (MBLK, B), 1)
    moh = (bvec == iota_b).astype(F32)
    islast = (bvec != bn_ref[...]).astype(F32)
    v1 = _dot_c0(moh * islast, hn)
    cnt = jnp.broadcast_to(jnp.sum(moh, axis=0, keepdims=True), (8, B))

    @pl.when(i == 0)
    def _():
        v1_ref[...] = v1
        cnt_ref[...] = cnt

    @pl.when(i > 0)
    def _():
        v1_ref[...] += v1
        cnt_ref[...] += cnt


def _tc_gru_vn(h, aggh, gg, wihT, whhT, bih, bhh, batch2d, batchnext2d):
    zero = lambda i: (0, 0)
    return pl.pallas_call(
        _tc_gru_vn_body,
        grid=(N // MBLK,),
        in_specs=[pl.BlockSpec((MBLK, H), lambda i: (i, 0)),
                  pl.BlockSpec((MBLK, H), lambda i: (i, 0)),
                  pl.BlockSpec((H, H), zero),
                  pl.BlockSpec((H, 3 * H), zero),
                  pl.BlockSpec((H, 3 * H), zero),
                  pl.BlockSpec((1, 3 * H), zero),
                  pl.BlockSpec((1, 3 * H), zero),
                  pl.BlockSpec((MBLK, 1), lambda i: (i, 0)),
                  pl.BlockSpec((MBLK, 1), lambda i: (i, 0))],
        out_specs=[pl.BlockSpec((MBLK, H), lambda i: (i, 0)),
                   pl.BlockSpec((B, H), zero),
                   pl.BlockSpec((8, B), zero)],
        out_shape=[jax.ShapeDtypeStruct((N, H), F32),
                   jax.ShapeDtypeStruct((B, H), F32),
                   jax.ShapeDtypeStruct((8, B), F32)],
    )(h, aggh, gg, wihT, whhT, bih, bhh, batch2d, batchnext2d)


RBLK = 2048


def _fixup_vn(v1, cnt_row, hlast):
    ii = lax.broadcasted_iota(jnp.int32, (B, B), 0)
    bb = lax.broadcasted_iota(jnp.int32, (B, B), 1)
    eye = (ii == bb).astype(F32)
    cand = jnp.where(
        cnt_row > 0.0,
        lax.broadcasted_iota(jnp.int32, (1, B), 1).astype(F32), -1.0)
    candT = lax.dot_general(eye, cand, (((1,), (1,)), ((), ())),
                            preferred_element_type=F32,
                            precision=HIGH)
    jmat = jnp.where(ii <= bb, jnp.broadcast_to(candT, (B, B)), -1.0)
    jrow = jnp.max(jmat, axis=0, keepdims=True)
    onehot = (ii == jrow.astype(jnp.int32)).astype(F32)
    v_n = _dot_c0(onehot, v1)
    jT = lax.dot_general(eye, jrow, (((1,), (1,)), ((), ())),
                         preferred_element_type=F32, precision=HIGH)
    return jnp.where(jT < 0.0, jnp.broadcast_to(hlast, (B, H)), v_n)


ZBLK = 16384
RSTEPS = N // RBLK


def _tc_readout_logits_body(h_ref, b_ref, v1_ref, cnt_ref, hlast_ref,
                            w1T_ref, w2T_ref, b12_ref, qw_ref, qb_ref,
                            w3aT_ref, w3bT_ref, b3_ref, emb_ref, z_ref,
                            sh_ref):
    i = pl.program_id(0)

    @pl.when(i < RSTEPS)
    def _():
        v_n = _fixup_vn(v1_ref[...], cnt_ref[0:1, :], hlast_ref[...])
        h = h_ref[...]
        bvec = b_ref[...]
        iota_b = lax.broadcasted_iota(jnp.int32, (RBLK, B), 1)
        moh = (bvec == iota_b).astype(F32)
        vrep = _dotT(moh, v_n)
        pre = (_dotD(vrep, w1T_ref[...]) + _dotD(h, w2T_ref[...])
               + b12_ref[...])
        sig = jax.nn.sigmoid(pre)
        alpha = (jnp.sum(sig * qw_ref[...], axis=1, keepdims=True)
                 + qb_ref[...][:, :1])
        s_g = _dot_c0(moh, alpha * h)
        part = _dotD(s_g, w3bT_ref[...])

        @pl.when(i == 0)
        def _():
            sh_ref[...] = _dotD(v_n, w3aT_ref[...]) + b3_ref[...] + part

        @pl.when(i > 0)
        def _():
            sh_ref[...] += part

    @pl.when(i >= RSTEPS)
    def _():
        z_ref[...] = lax.dot_general(
            sh_ref[...], emb_ref[...], (((1,), (1,)), ((), ())),
            preferred_element_type=F32)


def _tc_readout_logits(h, batch2d, v1, cnt, hlast, w1T, w2T, b12, qw, qb,
                       w3aT, w3bT, b3, emb):
    n_vocab = emb.shape[0]
    zsteps = (n_vocab + ZBLK - 1) // ZBLK
    zero = lambda i: (0, 0)
    hidx = lambda i: (jnp.minimum(i, RSTEPS - 1), 0)
    zidx = lambda i: (0, jnp.maximum(i - RSTEPS, 0))
    eidx = lambda i: (jnp.maximum(i - RSTEPS, 0), 0)
    return pl.pallas_call(
        _tc_readout_logits_body,
        grid=(RSTEPS + zsteps,),
        in_specs=[pl.BlockSpec((RBLK, H), hidx),
                  pl.BlockSpec((RBLK, 1), hidx),
                  pl.BlockSpec((B, H), zero),
                  pl.BlockSpec((8, B), zero),
                  pl.BlockSpec((1, H), zero),
                  pl.BlockSpec((H, H), zero),
                  pl.BlockSpec((H, H), zero),
                  pl.BlockSpec((1, H), zero),
                  pl.BlockSpec((1, H), zero),
                  pl.BlockSpec((1, H), zero),
                  pl.BlockSpec((H, H), zero),
                  pl.BlockSpec((H, H), zero),
                  pl.BlockSpec((1, H), zero),
                  pl.BlockSpec((ZBLK, H), eidx)],
        out_specs=pl.BlockSpec((B, ZBLK), zidx),
        out_shape=jax.ShapeDtypeStruct((B, n_vocab), F32),
        scratch_shapes=[pltpu.VMEM((B, H), F32)],
    )(h, batch2d, v1, cnt, hlast, w1T, w2T, b12, qw, qb, w3aT, w3bT, b3,
      emb)


def kernel(x, edge_index, batch, edge_attr, emb, gg_weight, w_ih, w_hh,
           b_ih, b_hh, W1, b1, W2, b2, q_w, q_b, W3, b3):
    del edge_attr
    xm1 = (x - 1).reshape(N // GCH, GCH)
    src2d = edge_index[0].reshape(E // ECH, ECH)
    dst2d = edge_index[1].reshape(E // ECH, ECH)

    wihT = w_ih.T
    whhT = w_hh.T
    bih = b_ih.reshape(1, 3 * H)
    bhh = b_hh.reshape(1, 3 * H)

    h0 = _sc_gather_fn()(emb, xm1)

    batch2d = batch.astype(jnp.int32).reshape(N, 1)
    batchnext2d = jnp.concatenate(
        [batch2d[1:], jnp.full((1, 1), -1, jnp.int32)], axis=0)

    aggh0 = _sc_edge_agg_fn()(h0, src2d, dst2d)
    h1 = _tc_gru(h0, aggh0, gg_weight[0], wihT, whhT, bih, bhh)
    aggh1 = _sc_edge_agg_fn()(h1, src2d, dst2d)
    h2, v1, cnt = _tc_gru_vn(h1, aggh1, gg_weight[1], wihT, whhT, bih, bhh,
                             batch2d, batchnext2d)

    return _tc_readout_logits(
        h2, batch2d, v1, cnt, h2[N - 1:N],
        W1.T, W2.T, (b1 + b2).reshape(1, H),
        q_w.reshape(1, H), jnp.broadcast_to(q_b.reshape(1, 1), (1, H)),
        W3[:, :H].T, W3[:, H:].T, b3.reshape(1, H), emb)


# speedup vs baseline: 4.6118x; 1.0023x over previous
import functools

import jax
import jax.numpy as jnp
from jax import lax
from jax.experimental import pallas as pl
from jax.experimental.pallas import tpu as pltpu
from jax.experimental.pallas import tpu_sc as plsc

H = 128
N = 16384
E = 65536
B = 16
NC, NS, LANES = 2, 16, 16
NW = NC * NS

F32 = jnp.float32
HIGH = lax.Precision.HIGHEST


def _dotT(a, b):
    return jnp.dot(a, b, preferred_element_type=F32, precision=HIGH)


def _dotD(a, b):
    return jnp.dot(a, b, preferred_element_type=F32)


def _dot_c0(a, b):
    return lax.dot_general(a, b, (((0,), (0,)), ((), ())),
                           preferred_element_type=F32, precision=HIGH)


ROWS_PER_W = N // NW
GCH = 128
GITER = ROWS_PER_W // GCH

_SC_MESH_KW = dict(core_axis_name="c", subcore_axis_name="s",
                   num_cores=NC, num_subcores=NS)


@functools.cache
def _sc_mesh():
    return plsc.VectorSubcoreMesh(**_SC_MESH_KW)


@functools.cache
def _sc_gather_fn():
    @functools.partial(
        pl.kernel,
        out_type=jax.ShapeDtypeStruct((N, H), F32),
        mesh=_sc_mesh(),
        scratch_types=[
            pltpu.VMEM((GITER, GCH), jnp.int32),
            pltpu.VMEM((GCH, H), F32),
            pltpu.VMEM((GCH, H), F32),
            pltpu.SemaphoreType.DMA,
            pltpu.SemaphoreType.DMA,
        ],
    )
    def _sc_gather(table_hbm, idx_hbm, out_hbm, idx_v, rows0, rows1,
                   sem0, sem1):
        wid = lax.axis_index("s") * NC + lax.axis_index("c")
        base = wid * ROWS_PER_W
        pltpu.sync_copy(idx_hbm.at[pl.ds(wid * GITER, GITER)], idx_v)
        bufs = (rows0, rows1)
        sems = (sem0, sem1)
        pltpu.async_copy(table_hbm.at[idx_v.at[0]], rows0, sem0)
        for j in range(GITER):
            b = j % 2
            if j + 1 < GITER:
                pltpu.async_copy(table_hbm.at[idx_v.at[j + 1]],
                                 bufs[1 - b], sems[1 - b])
            pltpu.make_async_copy(table_hbm.at[idx_v.at[j]],
                                  bufs[b], sems[b]).wait()
            pltpu.sync_copy(bufs[b], out_hbm.at[pl.ds(base + j * GCH, GCH)])

    return _sc_gather


NHALF = N // NC
E_PER_TILE = E // NS
ECH = 128
EITER = E_PER_TILE // ECH
OUT_ROWS_PER_TILE = NHALF // NS


@functools.cache
def _sc_edge_agg_fn():
    @functools.partial(
        pl.kernel,
        out_type=jax.ShapeDtypeStruct((N, H), F32),
        mesh=_sc_mesh(),
        scratch_types=[
            pltpu.VMEM((EITER, ECH), jnp.int32),
            pltpu.VMEM((EITER, ECH), jnp.int32),
            pltpu.VMEM((ECH, H), F32),
            pltpu.VMEM((ECH, H), F32),
            pltpu.VMEM((ECH, H), F32),
            pltpu.VMEM_SHARED((NHALF + ECH, H), F32),
            pltpu.SemaphoreType.DMA,
            pltpu.SemaphoreType.DMA,
            pltpu.SemaphoreType.DMA,
        ],
    )
    def _sc_edge_agg(h_hbm, src_hbm, dst_hbm, out_hbm,
                     srcv, dstv, rows0, rows1, rows2, agg_sh,
                     sem0, sem1, sem2):
        c = lax.axis_index("c")
        s = lax.axis_index("s")

        pltpu.sync_copy(src_hbm.at[pl.ds(s * EITER, EITER)], srcv)
        pltpu.sync_copy(dst_hbm.at[pl.ds(s * EITER, EITER)], dstv)

        rbase = c * NHALF

        def _locrow(j, carry):
            for q in range(ECH // LANES):
                d = dstv[j, pl.ds(q * LANES, LANES)] - rbase
                bad = (d < 0) | (d >= NHALF)
                dstv[j, pl.ds(q * LANES, LANES)] = jnp.where(bad, NHALF, d)
            return carry
        lax.fori_loop(0, EITER, _locrow, 0)

        def _zrow(i, carry):
            for q in range(H // LANES):
                rows0[i, pl.ds(q * LANES, LANES)] = jnp.zeros((LANES,), F32)
            return carry
        lax.fori_loop(0, ECH, _zrow, 0)
        for j in range(OUT_ROWS_PER_TILE // ECH):
            pltpu.sync_copy(
                rows0,
                agg_sh.at[pl.ds(s * OUT_ROWS_PER_TILE + j * ECH, ECH)])

        @pl.when(s == 0)
        def _():
            pltpu.sync_copy(rows0, agg_sh.at[pl.ds(NHALF, ECH)])

        plsc.subcore_barrier()

        bufs = (rows0, rows1, rows2)
        sems = (sem0, sem1, sem2)
        for j in range(2):
            pltpu.async_copy(h_hbm.at[srcv.at[j]], bufs[j], sems[j])
        for j in range(EITER):
            b = j % 3
            if j + 2 < EITER:
                nb = (j + 2) % 3
                pltpu.async_copy(h_hbm.at[srcv.at[j + 2]], bufs[nb],
                                 sems[nb])
            pltpu.make_async_copy(h_hbm.at[srcv.at[j]],
                                  bufs[b], sems[b]).wait()
            pltpu.sync_copy(bufs[b], agg_sh.at[dstv.at[j]], add=True)
        plsc.subcore_barrier()

        pltpu.sync_copy(
            agg_sh.at[pl.ds(s * OUT_ROWS_PER_TILE, OUT_ROWS_PER_TILE)],
            out_hbm.at[pl.ds(rbase + s * OUT_ROWS_PER_TILE,
                             OUT_ROWS_PER_TILE)])

    return _sc_edge_agg


MBLK = 4096


def _gru_math(h, aggh, gg_ref, wihT_ref, whhT_ref, bih_ref, bhh_ref):
    wA = _dotD(gg_ref[...], wihT_ref[...])
    gi = _dotD(aggh, wA) + bih_ref[...]
    gh = _dotD(h, whhT_ref[...]) + bhh_ref[...]
    i_r, i_z, i_n = gi[:, :H], gi[:, H:2 * H], gi[:, 2 * H:]
    h_r, h_z, h_n = gh[:, :H], gh[:, H:2 * H], gh[:, 2 * H:]
    r = jax.nn.sigmoid(i_r + h_r)
    z = jax.nn.sigmoid(i_z + h_z)
    n = jnp.tanh(i_n + r * h_n)
    return (1.0 - z) * n + z * h


def _tc_gru_body(h_ref, agg_ref, gg_ref, wihT_ref, whhT_ref, bih_ref,
                 bhh_ref, hout_ref):
    hout_ref[...] = _gru_math(h_ref[...], agg_ref[...], gg_ref, wihT_ref,
                              whhT_ref, bih_ref, bhh_ref)


def _tc_gru(h, aggh, gg, wihT, whhT, bih, bhh):
    zero = lambda i: (0, 0)
    return pl.pallas_call(
        _tc_gru_body,
        grid=(N // MBLK,),
        in_specs=[pl.BlockSpec((MBLK, H), lambda i: (i, 0)),
                  pl.BlockSpec((MBLK, H), lambda i: (i, 0)),
                  pl.BlockSpec((H, H), zero),
                  pl.BlockSpec((H, 3 * H), zero),
                  pl.BlockSpec((H, 3 * H), zero),
                  pl.BlockSpec((1, 3 * H), zero),
                  pl.BlockSpec((1, 3 * H), zero)],
        out_specs=pl.BlockSpec((MBLK, H), lambda i: (i, 0)),
        out_shape=jax.ShapeDtypeStruct((N, H), F32),
    )(h, aggh, gg, wihT, whhT, bih, bhh)


def _tc_gru_vn_body(h_ref, agg_ref, gg_ref, wihT_ref, whhT_ref, bih_ref,
                    bhh_ref, b_ref, bn_ref, hout_ref, v1_ref, cnt_ref):
    i = pl.program_id(0)
    hn = _gru_math(h_ref[...], agg_ref[...], gg_ref, wihT_ref,
                   whhT_ref, bih_ref, bhh_ref)
    hout_ref[...] = hn
    bvec = b_ref[...]
    iota_b = lax.broadcasted_iota(jnp.int32, (MBLK, B), 1)
    moh = (bvec == iota_b).astype(F32)
    islast = (bvec != bn_ref[...]).astype(F32)
    v1 = _dot_c0(moh * islast, hn)
    cnt = jnp.broadcast_to(jnp.sum(moh, axis=0, keepdims=True), (8, B))

    @pl.when(i == 0)
    def _():
        v1_ref[...] = v1
        cnt_ref[...] = cnt

    @pl.when(i > 0)
    def _():
        v1_ref[...] += v1
        cnt_ref[...] += cnt


def _tc_gru_vn(h, aggh, gg, wihT, whhT, bih, bhh, batch2d, batchnext2d):
    zero = lambda i: (0, 0)
    return pl.pallas_call(
        _tc_gru_vn_body,
        grid=(N // MBLK,),
        in_specs=[pl.BlockSpec((MBLK, H), lambda i: (i, 0)),
                  pl.BlockSpec((MBLK, H), lambda i: (i, 0)),
                  pl.BlockSpec((H, H), zero),
                  pl.BlockSpec((H, 3 * H), zero),
                  pl.BlockSpec((H, 3 * H), zero),
                  pl.BlockSpec((1, 3 * H), zero),
                  pl.BlockSpec((1, 3 * H), zero),
                  pl.BlockSpec((MBLK, 1), lambda i: (i, 0)),
                  pl.BlockSpec((MBLK, 1), lambda i: (i, 0))],
        out_specs=[pl.BlockSpec((MBLK, H), lambda i: (i, 0)),
                   pl.BlockSpec((B, H), zero),
                   pl.BlockSpec((8, B), zero)],
        out_shape=[jax.ShapeDtypeStruct((N, H), F32),
                   jax.ShapeDtypeStruct((B, H), F32),
                   jax.ShapeDtypeStruct((8, B), F32)],
    )(h, aggh, gg, wihT, whhT, bih, bhh, batch2d, batchnext2d)


RBLK = 2048


def _fixup_vn(v1, cnt_row, hlast):
    ii = lax.broadcasted_iota(jnp.int32, (B, B), 0)
    bb = lax.broadcasted_iota(jnp.int32, (B, B), 1)
    eye = (ii == bb).astype(F32)
    cand = jnp.where(
        cnt_row > 0.0,
        lax.broadcasted_iota(jnp.int32, (1, B), 1).astype(F32), -1.0)
    candT = lax.dot_general(eye, cand, (((1,), (1,)), ((), ())),
                            preferred_element_type=F32,
                            precision=HIGH)
    jmat = jnp.where(ii <= bb, jnp.broadcast_to(candT, (B, B)), -1.0)
    jrow = jnp.max(jmat, axis=0, keepdims=True)
    onehot = (ii == jrow.astype(jnp.int32)).astype(F32)
    v_n = _dot_c0(onehot, v1)
    jT = lax.dot_general(eye, jrow, (((1,), (1,)), ((), ())),
                         preferred_element_type=F32, precision=HIGH)
    return jnp.where(jT < 0.0, jnp.broadcast_to(hlast, (B, H)), v_n)


ZBLK = 16384
RSTEPS = N // RBLK


def _tc_readout_logits_body(h_ref, b_ref, v1_ref, cnt_ref, hlast_ref,
                            w1T_ref, w2T_ref, b12_ref, qw_ref, qb_ref,
                            w3aT_ref, w3bT_ref, b3_ref, emb_ref, z_ref,
                            sh_ref):
    i = pl.program_id(0)

    @pl.when(i < RSTEPS)
    def _():
        v_n = _fixup_vn(v1_ref[...], cnt_ref[0:1, :], hlast_ref[...])
        h = h_ref[...]
        bvec = b_ref[...]
        iota_b = lax.broadcasted_iota(jnp.int32, (RBLK, B), 1)
        moh = (bvec == iota_b).astype(F32)
        vrep = _dotT(moh, v_n)
        pre = (_dotD(vrep, w1T_ref[...]) + _dotD(h, w2T_ref[...])
               + b12_ref[...])
        sig = jax.nn.sigmoid(pre)
        alpha = (jnp.sum(sig * qw_ref[...], axis=1, keepdims=True)
                 + qb_ref[...][:, :1])
        s_g = _dot_c0(moh, alpha * h)
        part = _dotD(s_g, w3bT_ref[...])

        @pl.when(i == 0)
        def _():
            sh_ref[...] = _dotD(v_n, w3aT_ref[...]) + b3_ref[...] + part

        @pl.when(i > 0)
        def _():
            sh_ref[...] += part

    @pl.when(i >= RSTEPS)
    def _():
        z_ref[...] = lax.dot_general(
            sh_ref[...], emb_ref[...], (((1,), (1,)), ((), ())),
            preferred_element_type=F32)


def _tc_readout_logits(h, batch2d, v1, cnt, hlast, w1T, w2T, b12, qw, qb,
                       w3aT, w3bT, b3, emb):
    n_vocab = emb.shape[0]
    zsteps = (n_vocab + ZBLK - 1) // ZBLK
    zero = lambda i: (0, 0)
    hidx = lambda i: (jnp.minimum(i, RSTEPS - 1), 0)
    zidx = lambda i: (0, jnp.maximum(i - RSTEPS, 0))
    eidx = lambda i: (jnp.maximum(i - RSTEPS, 0), 0)
    return pl.pallas_call(
        _tc_readout_logits_body,
        grid=(RSTEPS + zsteps,),
        in_specs=[pl.BlockSpec((RBLK, H), hidx),
                  pl.BlockSpec((RBLK, 1), hidx),
                  pl.BlockSpec((B, H), zero),
                  pl.BlockSpec((8, B), zero),
                  pl.BlockSpec((1, H), zero),
                  pl.BlockSpec((H, H), zero),
                  pl.BlockSpec((H, H), zero),
                  pl.BlockSpec((1, H), zero),
                  pl.BlockSpec((1, H), zero),
                  pl.BlockSpec((1, H), zero),
                  pl.BlockSpec((H, H), zero),
                  pl.BlockSpec((H, H), zero),
                  pl.BlockSpec((1, H), zero),
                  pl.BlockSpec((ZBLK, H), eidx)],
        out_specs=pl.BlockSpec((B, ZBLK), zidx),
        out_shape=jax.ShapeDtypeStruct((B, n_vocab), F32),
        scratch_shapes=[pltpu.VMEM((B, H), F32)],
    )(h, batch2d, v1, cnt, hlast, w1T, w2T, b12, qw, qb, w3aT, w3bT, b3,
      emb)


def kernel(x, edge_index, batch, edge_attr, emb, gg_weight, w_ih, w_hh,
           b_ih, b_hh, W1, b1, W2, b2, q_w, q_b, W3, b3):
    del edge_attr
    xm1 = (x - 1).reshape(N // GCH, GCH)
    src2d = edge_index[0].reshape(E // ECH, ECH)
    dst2d = edge_index[1].reshape(E // ECH, ECH)

    wihT = w_ih.T
    whhT = w_hh.T
    bih = b_ih.reshape(1, 3 * H)
    bhh = b_hh.reshape(1, 3 * H)

    h0 = _sc_gather_fn()(emb, xm1)

    batch2d = batch.astype(jnp.int32).reshape(N, 1)
    batchnext2d = jnp.concatenate(
        [batch2d[1:], jnp.full((1, 1), -1, jnp.int32)], axis=0)

    aggh0 = _sc_edge_agg_fn()(h0, src2d, dst2d)
    h1 = _tc_gru(h0, aggh0, gg_weight[0], wihT, whhT, bih, bhh)
    aggh1 = _sc_edge_agg_fn()(h1, src2d, dst2d)
    h2, v1, cnt = _tc_gru_vn(h1, aggh1, gg_weight[1], wihT, whhT, bih, bhh,
                             batch2d, batchnext2d)

    return _tc_readout_logits(
        h2, batch2d, v1, cnt, h2[N - 1:N],
        W1.T, W2.T, (b1 + b2).reshape(1, H),
        q_w.reshape(1, H), jnp.broadcast_to(q_b.reshape(1, 1), (1, H)),
        W3[:, :H].T, W3[:, H:].T, b3.reshape(1, H), emb)
